# baseline (device time: 42622 ns/iter reference)
import jax
import jax.numpy as jnp
from jax import lax
from jax.experimental import pallas as pl
from jax.experimental.pallas import tpu as pltpu

N_DEV = 4


def kernel(x, Wq, K_ext, V_ext, Wo):
    B, Sq, D = x.shape
    _, Skv, Hq, Dh = K_ext.shape
    HH = Hq // 2
    NS = B * Hq

    Kt = jnp.transpose(K_ext, (0, 2, 1, 3))
    Vt = jnp.transpose(V_ext, (0, 2, 1, 3))
    Wqh = jnp.transpose(Wq.reshape(D, Hq, Dh), (1, 0, 2))

    def slot_bh(slot):
        half, r = divmod(slot, B * HH)
        b, hh = divmod(r, HH)
        return b, hh + half * HH

    A = list(range(B * HH))
    Bs = list(range(B * HH, NS))

    def body(x_ref, wq_ref, k_ref, v_ref, wo_ref, out_ref,
             cbuf, c_send, c_recv, f_send):
        my = lax.axis_index("i")
        left = lax.rem(my + N_DEV - 1, N_DEV)
        right = lax.rem(my + 1, N_DEV)

        def rcopy(src, dst, ssem, rsem, dev):
            return pltpu.make_async_remote_copy(
                src_ref=src, dst_ref=dst, send_sem=ssem, recv_sem=rsem,
                device_id=(dev,), device_id_type=pl.DeviceIdType.MESH,
            )

        barrier_sem = pltpu.get_barrier_semaphore()
        for nbr in (left, right):
            pl.semaphore_signal(
                barrier_sem, inc=1,
                device_id=(nbr,), device_id_type=pl.DeviceIdType.MESH,
            )
        pl.semaphore_wait(barrier_sem, 2)

        @pl.when(my == 0)
        def _():
            qb = lax.broadcasted_iota(jnp.int32, (Sq, Skv), 0) // 64
            kb = lax.broadcasted_iota(jnp.int32, (Sq, Skv), 1) // 64
            mask = kb <= qb
            X2 = jnp.concatenate([x_ref[b] for b in range(B)], axis=0)
            qh = {}
            ctxs = {}
            order = []
            for k in range(B * HH):
                order += [A[k], Bs[k]]
            for slot in order:
                b, h = slot_bh(slot)
                if h not in qh:
                    qh[h] = jnp.dot(X2, wq_ref[h],
                                    preferred_element_type=jnp.float32)
                q = qh[h][b * Sq:(b + 1) * Sq]
                s = lax.dot_general(
                    q, k_ref[b, h], (((1,), (1,)), ((), ())),
                    preferred_element_type=jnp.float32,
                ) * 0.125
                e = jnp.exp(jnp.where(mask, s, -1e30))
                w = e / jnp.sum(e, axis=1, keepdims=True)
                ctx = jnp.dot(w, v_ref[b, h],
                              preferred_element_type=jnp.float32)
                ctxs[slot] = ctx
                cbuf[slot] = ctx.astype(jnp.bfloat16)
                tgt = 1 if slot in A else 3
                rcopy(cbuf.at[slot], cbuf.at[slot],
                      c_send.at[slot], c_recv.at[slot], tgt).start()
            full = jnp.concatenate(
                [jnp.concatenate(
                    [ctxs[slot] for slot in sorted(
                        range(NS), key=lambda s_: slot_bh(s_)[1])
                     if slot_bh(slot)[0] == b], axis=1)
                 for b in range(B)], axis=0)
            o = jnp.dot(full, wo_ref[...],
                        preferred_element_type=jnp.float32)
            for b in range(B):
                out_ref[b] = o[b * Sq:(b + 1) * Sq]
            for slot in range(NS):
                rcopy(cbuf.at[slot], cbuf.at[slot],
                      c_send.at[slot], c_recv.at[slot], right).wait_send()

        def consumer(sched):
            def _run():
                first_b = set(range(B))
                for (s0, s1), fwd in sched:
                    for slot in (s0, s1):
                        rcopy(cbuf.at[slot], cbuf.at[slot],
                              c_send.at[slot], c_recv.at[slot],
                              right).wait_recv()
                        if fwd is not None:
                            rcopy(cbuf.at[slot], cbuf.at[slot],
                                  f_send.at[slot], c_recv.at[slot],
                                  fwd).start()
                    b, h = slot_bh(s0)
                    m = jnp.concatenate(
                        [cbuf[s0], cbuf[s1]], axis=1).astype(jnp.float32)
                    contrib = jnp.dot(m, wo_ref[pl.ds(h * Dh, 2 * Dh)],
                                      preferred_element_type=jnp.float32)
                    if b in first_b:
                        out_ref[b] = contrib
                        first_b.discard(b)
                    else:
                        out_ref[b] = out_ref[b] + contrib
                for (s0, s1), fwd in sched:
                    if fwd is not None:
                        for slot in (s0, s1):
                            rcopy(cbuf.at[slot], cbuf.at[slot],
                                  f_send.at[slot], c_recv.at[slot],
                                  right).wait_send()
            return _run

        Ap = [(A[2 * i], A[2 * i + 1]) for i in range(len(A) // 2)]
        Bp = [(Bs[2 * i], Bs[2 * i + 1]) for i in range(len(Bs) // 2)]

        sched1 = [(Ap[0], 2), (Ap[1], 2), (Bp[0], None), (Ap[2], 2),
                  (Bp[1], None), (Ap[3], 2), (Bp[2], None), (Bp[3], None)]
        sched3 = [(Bp[0], 2), (Bp[1], 2), (Ap[0], None), (Bp[2], 2),
                  (Ap[1], None), (Bp[3], 2), (Ap[2], None), (Ap[3], None)]
        sched2 = []
        for i in range(len(Ap)):
            sched2 += [(Ap[i], 3), (Bp[i], 1)]

        pl.when(my == 1)(consumer(sched1))
        pl.when(my == 2)(consumer(sched2))
        pl.when(my == 3)(consumer(sched3))

    return pl.pallas_call(
        body,
        out_shape=jax.ShapeDtypeStruct((B, Sq, D), jnp.float32),
        in_specs=[pl.BlockSpec(memory_space=pltpu.VMEM)] * 5,
        out_specs=pl.BlockSpec(memory_space=pltpu.VMEM),
        scratch_shapes=[
            pltpu.VMEM((NS, Sq, Dh), jnp.bfloat16),
            pltpu.SemaphoreType.DMA((NS,)),
            pltpu.SemaphoreType.DMA((NS,)),
            pltpu.SemaphoreType.DMA((NS,)),
        ],
        compiler_params=pltpu.CompilerParams(collective_id=0),
    )(x, Wqh, Kt, Vt, Wo)


# device time: 40267 ns/iter; 1.0585x vs baseline; 1.0585x over previous
import jax
import jax.numpy as jnp
from jax import lax
from jax.experimental import pallas as pl
from jax.experimental.pallas import tpu as pltpu

N_DEV = 4


def kernel(x, Wq, K_ext, V_ext, Wo):
    B, Sq, D = x.shape
    _, Skv, Hq, Dh = K_ext.shape
    HH = Hq // 2
    NS = B * Hq

    Kt = jnp.transpose(K_ext, (0, 2, 1, 3))
    Vt = jnp.transpose(V_ext, (0, 2, 1, 3))
    Wqh = jnp.transpose(Wq.reshape(D, Hq, Dh), (1, 0, 2))

    def slot_bh(slot):
        half, r = divmod(slot, B * HH)
        b, hh = divmod(r, HH)
        return b, hh + half * HH

    A = list(range(B * HH))
    Bs = list(range(B * HH, NS))

    def body(x_ref, wq_ref, k_ref, v_ref, wo_ref, out_ref,
             cbuf, c_send, c_recv, f_send):
        my = lax.axis_index("i")
        left = lax.rem(my + N_DEV - 1, N_DEV)
        right = lax.rem(my + 1, N_DEV)

        def rcopy(src, dst, ssem, rsem, dev):
            return pltpu.make_async_remote_copy(
                src_ref=src, dst_ref=dst, send_sem=ssem, recv_sem=rsem,
                device_id=(dev,), device_id_type=pl.DeviceIdType.MESH,
            )

        barrier_sem = pltpu.get_barrier_semaphore()
        for nbr in (left, right):
            pl.semaphore_signal(
                barrier_sem, inc=1,
                device_id=(nbr,), device_id_type=pl.DeviceIdType.MESH,
            )
        pl.semaphore_wait(barrier_sem, 2)

        @pl.when(my == 0)
        def _():
            qb = lax.broadcasted_iota(jnp.int32, (Sq, Skv), 0) // 64
            kb = lax.broadcasted_iota(jnp.int32, (Sq, Skv), 1) // 64
            mask = kb <= qb
            X2 = jnp.concatenate([x_ref[b] for b in range(B)], axis=0)
            qh = {}
            ctxs = {}
            order = []
            for k in range(B * HH):
                order += [A[k], Bs[k]]
            for slot in order:
                b, h = slot_bh(slot)
                if h not in qh:
                    qh[h] = jnp.dot(X2, wq_ref[h],
                                    preferred_element_type=jnp.float32)
                q = qh[h][b * Sq:(b + 1) * Sq]
                s = lax.dot_general(
                    q, k_ref[b, h], (((1,), (1,)), ((), ())),
                    preferred_element_type=jnp.float32,
                ) * 0.125
                e = jnp.exp(jnp.where(mask, s, -1e30))
                w = e / jnp.sum(e, axis=1, keepdims=True)
                ctx = jnp.dot(w, v_ref[b, h],
                              preferred_element_type=jnp.float32)
                ctxs[slot] = ctx
                cbuf[slot] = ctx.astype(jnp.bfloat16)
                tgt = 1 if slot in A else 3
                rcopy(cbuf.at[slot], cbuf.at[slot],
                      c_send.at[slot], c_recv.at[slot], tgt).start()
            full = jnp.concatenate(
                [jnp.concatenate(
                    [ctxs[slot] for slot in sorted(
                        range(NS), key=lambda s_: slot_bh(s_)[1])
                     if slot_bh(slot)[0] == b], axis=1)
                 for b in range(B)], axis=0)
            o = jnp.dot(full, wo_ref[...],
                        preferred_element_type=jnp.float32)
            for b in range(B):
                out_ref[b] = o[b * Sq:(b + 1) * Sq]
            for slot in range(NS):
                rcopy(cbuf.at[slot], cbuf.at[slot],
                      c_send.at[slot], c_recv.at[slot], right).wait_send()

        def consumer(sched):
            def _run():
                for slot, fwd in sched:
                    rcopy(cbuf.at[slot], cbuf.at[slot],
                          c_send.at[slot], c_recv.at[slot],
                          right).wait_recv()
                    if fwd is not None:
                        rcopy(cbuf.at[slot], cbuf.at[slot],
                              f_send.at[slot], c_recv.at[slot],
                              fwd).start()
                full = jnp.concatenate(
                    [jnp.concatenate(
                        [cbuf[slot] for slot in sorted(
                            range(NS), key=lambda s_: slot_bh(s_)[1])
                         if slot_bh(slot)[0] == b], axis=1)
                     for b in range(B)], axis=0).astype(jnp.float32)
                o = jnp.dot(full, wo_ref[...],
                            preferred_element_type=jnp.float32)
                for b in range(B):
                    out_ref[b] = o[b * Sq:(b + 1) * Sq]
                for slot, fwd in sched:
                    if fwd is not None:
                        rcopy(cbuf.at[slot], cbuf.at[slot],
                              f_send.at[slot], c_recv.at[slot],
                              right).wait_send()
            return _run

        sched1 = []
        ai, bi = 0, 0
        for i in range(NS):
            if ai < len(A) and (ai - bi < 3 or bi >= len(Bs)):
                sched1.append((A[ai], 2))
                ai += 1
            else:
                sched1.append((Bs[bi], None))
                bi += 1
        sched3 = [(Bs[A.index(s)] if s in A else A[Bs.index(s)],
                   2 if f == 2 else None) for s, f in sched1]
        sched2 = []
        for k in range(B * HH):
            sched2 += [(A[k], 3), (Bs[k], 1)]

        pl.when(my == 1)(consumer(sched1))
        pl.when(my == 2)(consumer(sched2))
        pl.when(my == 3)(consumer(sched3))

    return pl.pallas_call(
        body,
        out_shape=jax.ShapeDtypeStruct((B, Sq, D), jnp.float32),
        in_specs=[pl.BlockSpec(memory_space=pltpu.VMEM)] * 5,
        out_specs=pl.BlockSpec(memory_space=pltpu.VMEM),
        scratch_shapes=[
            pltpu.VMEM((NS, Sq, Dh), jnp.bfloat16),
            pltpu.SemaphoreType.DMA((NS,)),
            pltpu.SemaphoreType.DMA((NS,)),
            pltpu.SemaphoreType.DMA((NS,)),
        ],
        compiler_params=pltpu.CompilerParams(collective_id=0),
    )(x, Wqh, Kt, Vt, Wo)


# device time: 39037 ns/iter; 1.0918x vs baseline; 1.0315x over previous
import jax
import jax.numpy as jnp
from jax import lax
from jax.experimental import pallas as pl
from jax.experimental.pallas import tpu as pltpu

N_DEV = 4


def kernel(x, Wq, K_ext, V_ext, Wo):
    B, Sq, D = x.shape
    _, Skv, Hq, Dh = K_ext.shape
    HH = Hq // 2
    NS = B * Hq

    Kt = jnp.transpose(K_ext.astype(jnp.bfloat16), (0, 2, 1, 3))
    Vt = jnp.transpose(V_ext.astype(jnp.bfloat16), (0, 2, 1, 3))
    Wqh = jnp.transpose(Wq.astype(jnp.bfloat16).reshape(D, Hq, Dh),
                        (1, 0, 2))

    def slot_bh(slot):
        half, r = divmod(slot, B * HH)
        b, hh = divmod(r, HH)
        return b, hh + half * HH

    A = list(range(B * HH))
    Bs = list(range(B * HH, NS))

    def body(x_ref, wq_ref, k_ref, v_ref, wo_ref, out_ref,
             cbuf, c_send, c_recv, f_send):
        my = lax.axis_index("i")
        left = lax.rem(my + N_DEV - 1, N_DEV)
        right = lax.rem(my + 1, N_DEV)

        def rcopy(src, dst, ssem, rsem, dev):
            return pltpu.make_async_remote_copy(
                src_ref=src, dst_ref=dst, send_sem=ssem, recv_sem=rsem,
                device_id=(dev,), device_id_type=pl.DeviceIdType.MESH,
            )

        barrier_sem = pltpu.get_barrier_semaphore()
        for nbr in (left, right):
            pl.semaphore_signal(
                barrier_sem, inc=1,
                device_id=(nbr,), device_id_type=pl.DeviceIdType.MESH,
            )
        pl.semaphore_wait(barrier_sem, 2)

        @pl.when(my == 0)
        def _():
            qb = lax.broadcasted_iota(jnp.int32, (Sq, Skv), 0) // 64
            kb = lax.broadcasted_iota(jnp.int32, (Sq, Skv), 1) // 64
            mask = kb <= qb
            X2 = jnp.concatenate(
                [x_ref[b] for b in range(B)], axis=0).astype(jnp.bfloat16)
            qh = {}
            ctxs = {}
            order = []
            for k in range(B * HH):
                order += [A[k], Bs[k]]
            for slot in order:
                b, h = slot_bh(slot)
                if h not in qh:
                    qh[h] = jnp.dot(
                        X2, wq_ref[h], preferred_element_type=jnp.float32,
                    ).astype(jnp.bfloat16)
                q = qh[h][b * Sq:(b + 1) * Sq]
                s = lax.dot_general(
                    q, k_ref[b, h], (((1,), (1,)), ((), ())),
                    preferred_element_type=jnp.float32,
                ) * 0.125
                e = jnp.exp(jnp.where(mask, s, -1e30))
                unnorm = jnp.dot(e.astype(jnp.bfloat16), v_ref[b, h],
                                 preferred_element_type=jnp.float32)
                ctx = unnorm / jnp.sum(e, axis=1, keepdims=True)
                ctxs[slot] = ctx
                cbuf[slot] = ctx.astype(jnp.bfloat16)
                tgt = 1 if slot in A else 3
                rcopy(cbuf.at[slot], cbuf.at[slot],
                      c_send.at[slot], c_recv.at[slot], tgt).start()
            full = jnp.concatenate(
                [jnp.concatenate(
                    [ctxs[slot] for slot in sorted(
                        range(NS), key=lambda s_: slot_bh(s_)[1])
                     if slot_bh(slot)[0] == b], axis=1)
                 for b in range(B)], axis=0)
            o = jnp.dot(full, wo_ref[...],
                        preferred_element_type=jnp.float32)
            for b in range(B):
                out_ref[b] = o[b * Sq:(b + 1) * Sq]
            for slot in range(NS):
                rcopy(cbuf.at[slot], cbuf.at[slot],
                      c_send.at[slot], c_recv.at[slot], right).wait_send()

        def consumer(sched):
            def _run():
                for slot, fwd in sched:
                    rcopy(cbuf.at[slot], cbuf.at[slot],
                          c_send.at[slot], c_recv.at[slot],
                          right).wait_recv()
                    if fwd is not None:
                        rcopy(cbuf.at[slot], cbuf.at[slot],
                              f_send.at[slot], c_recv.at[slot],
                              fwd).start()
                full = jnp.concatenate(
                    [jnp.concatenate(
                        [cbuf[slot] for slot in sorted(
                            range(NS), key=lambda s_: slot_bh(s_)[1])
                         if slot_bh(slot)[0] == b], axis=1)
                     for b in range(B)], axis=0).astype(jnp.float32)
                o = jnp.dot(full, wo_ref[...],
                            preferred_element_type=jnp.float32)
                for b in range(B):
                    out_ref[b] = o[b * Sq:(b + 1) * Sq]
                for slot, fwd in sched:
                    if fwd is not None:
                        rcopy(cbuf.at[slot], cbuf.at[slot],
                              f_send.at[slot], c_recv.at[slot],
                              right).wait_send()
            return _run

        sched1 = []
        ai, bi = 0, 0
        for i in range(NS):
            if ai < len(A) and (ai - bi < 3 or bi >= len(Bs)):
                sched1.append((A[ai], 2))
                ai += 1
            else:
                sched1.append((Bs[bi], None))
                bi += 1
        sched3 = [(Bs[A.index(s)] if s in A else A[Bs.index(s)],
                   2 if f == 2 else None) for s, f in sched1]
        sched2 = []
        for k in range(B * HH):
            sched2 += [(A[k], 3), (Bs[k], 1)]

        pl.when(my == 1)(consumer(sched1))
        pl.when(my == 2)(consumer(sched2))
        pl.when(my == 3)(consumer(sched3))

    return pl.pallas_call(
        body,
        out_shape=jax.ShapeDtypeStruct((B, Sq, D), jnp.float32),
        in_specs=[pl.BlockSpec(memory_space=pltpu.VMEM)] * 5,
        out_specs=pl.BlockSpec(memory_space=pltpu.VMEM),
        scratch_shapes=[
            pltpu.VMEM((NS, Sq, Dh), jnp.bfloat16),
            pltpu.SemaphoreType.DMA((NS,)),
            pltpu.SemaphoreType.DMA((NS,)),
            pltpu.SemaphoreType.DMA((NS,)),
        ],
        compiler_params=pltpu.CompilerParams(collective_id=0),
    )(x, Wqh, Kt, Vt, Wo)


# device time: 38234 ns/iter; 1.1148x vs baseline; 1.0210x over previous
import jax
import jax.numpy as jnp
from jax import lax
from jax.experimental import pallas as pl
from jax.experimental.pallas import tpu as pltpu

N_DEV = 4


def kernel(x, Wq, K_ext, V_ext, Wo):
    B, Sq, D = x.shape
    _, Skv, Hq, Dh = K_ext.shape
    HH = Hq // 2
    NS = B * Hq

    Kt = jnp.transpose(K_ext.astype(jnp.bfloat16), (0, 2, 1, 3))
    Vt = jnp.transpose(V_ext.astype(jnp.bfloat16), (0, 2, 1, 3))
    Wqh = jnp.transpose(Wq.astype(jnp.bfloat16).reshape(D, Hq, Dh),
                        (1, 0, 2))

    def slot_bh(slot):
        half, r = divmod(slot, B * HH)
        b, hh = divmod(r, HH)
        return b, hh + half * HH

    A = list(range(B * HH))
    Bs = list(range(B * HH, NS))

    def body(x_ref, wq_ref, k_ref, v_ref, wo_ref, out_ref,
             cbuf, c_send, c_recv, f_send):
        my = lax.axis_index("i")
        left = lax.rem(my + N_DEV - 1, N_DEV)
        right = lax.rem(my + 1, N_DEV)

        def rcopy(src, dst, ssem, rsem, dev):
            return pltpu.make_async_remote_copy(
                src_ref=src, dst_ref=dst, send_sem=ssem, recv_sem=rsem,
                device_id=(dev,), device_id_type=pl.DeviceIdType.MESH,
            )

        barrier_sem = pltpu.get_barrier_semaphore()
        for nbr in (left, right):
            pl.semaphore_signal(
                barrier_sem, inc=1,
                device_id=(nbr,), device_id_type=pl.DeviceIdType.MESH,
            )
        pl.semaphore_wait(barrier_sem, 2)

        @pl.when(my == 0)
        def _():
            H2 = Sq // 2
            qb_t = lax.broadcasted_iota(jnp.int32, (H2, H2), 0) // 64
            kb_t = lax.broadcasted_iota(jnp.int32, (H2, H2), 1) // 64
            mask_t = kb_t <= qb_t
            qb_b = (lax.broadcasted_iota(jnp.int32, (H2, Skv), 0) + H2) // 64
            kb_b = lax.broadcasted_iota(jnp.int32, (H2, Skv), 1) // 64
            mask_b = kb_b <= qb_b
            X2 = jnp.concatenate(
                [x_ref[b] for b in range(B)], axis=0).astype(jnp.bfloat16)
            qh = {}
            ctxs = {}
            order = []
            for k in range(B * HH):
                order += [A[k], Bs[k]]
            for slot in order:
                b, h = slot_bh(slot)
                if h not in qh:
                    qh[h] = jnp.dot(
                        X2, wq_ref[h], preferred_element_type=jnp.float32,
                    ).astype(jnp.bfloat16)
                q = qh[h][b * Sq:(b + 1) * Sq]
                k = k_ref[b, h]
                v = v_ref[b, h]
                s_t = lax.dot_general(
                    q[:H2], k[:H2], (((1,), (1,)), ((), ())),
                    preferred_element_type=jnp.float32,
                ) * 0.125
                e_t = jnp.exp(jnp.where(mask_t, s_t, -1e30))
                un_t = jnp.dot(e_t.astype(jnp.bfloat16), v[:H2],
                               preferred_element_type=jnp.float32)
                s_b = lax.dot_general(
                    q[H2:], k, (((1,), (1,)), ((), ())),
                    preferred_element_type=jnp.float32,
                ) * 0.125
                e_b = jnp.exp(jnp.where(mask_b, s_b, -1e30))
                un_b = jnp.dot(e_b.astype(jnp.bfloat16), v,
                               preferred_element_type=jnp.float32)
                ctx = jnp.concatenate(
                    [un_t / jnp.sum(e_t, axis=1, keepdims=True),
                     un_b / jnp.sum(e_b, axis=1, keepdims=True)], axis=0)
                ctxs[slot] = ctx
                cbuf[slot] = ctx.astype(jnp.bfloat16)
                tgt = 1 if slot in A else 3
                rcopy(cbuf.at[slot], cbuf.at[slot],
                      c_send.at[slot], c_recv.at[slot], tgt).start()
            full = jnp.concatenate(
                [jnp.concatenate(
                    [ctxs[slot] for slot in sorted(
                        range(NS), key=lambda s_: slot_bh(s_)[1])
                     if slot_bh(slot)[0] == b], axis=1)
                 for b in range(B)], axis=0)
            o = jnp.dot(full, wo_ref[...],
                        preferred_element_type=jnp.float32)
            for b in range(B):
                out_ref[b] = o[b * Sq:(b + 1) * Sq]
            for slot in range(NS):
                rcopy(cbuf.at[slot], cbuf.at[slot],
                      c_send.at[slot], c_recv.at[slot], right).wait_send()

        def consumer(sched):
            def _run():
                for slot, fwd in sched:
                    rcopy(cbuf.at[slot], cbuf.at[slot],
                          c_send.at[slot], c_recv.at[slot],
                          right).wait_recv()
                    if fwd is not None:
                        rcopy(cbuf.at[slot], cbuf.at[slot],
                              f_send.at[slot], c_recv.at[slot],
                              fwd).start()
                full = jnp.concatenate(
                    [jnp.concatenate(
                        [cbuf[slot] for slot in sorted(
                            range(NS), key=lambda s_: slot_bh(s_)[1])
                         if slot_bh(slot)[0] == b], axis=1)
                     for b in range(B)], axis=0).astype(jnp.float32)
                o = jnp.dot(full, wo_ref[...],
                            preferred_element_type=jnp.float32)
                for b in range(B):
                    out_ref[b] = o[b * Sq:(b + 1) * Sq]
                for slot, fwd in sched:
                    if fwd is not None:
                        rcopy(cbuf.at[slot], cbuf.at[slot],
                              f_send.at[slot], c_recv.at[slot],
                              right).wait_send()
            return _run

        sched1 = []
        ai, bi = 0, 0
        for i in range(NS):
            if ai < len(A) and (ai - bi < 3 or bi >= len(Bs)):
                sched1.append((A[ai], 2))
                ai += 1
            else:
                sched1.append((Bs[bi], None))
                bi += 1
        sched3 = [(Bs[A.index(s)] if s in A else A[Bs.index(s)],
                   2 if f == 2 else None) for s, f in sched1]
        sched2 = []
        for k in range(B * HH):
            sched2 += [(A[k], 3), (Bs[k], 1)]

        pl.when(my == 1)(consumer(sched1))
        pl.when(my == 2)(consumer(sched2))
        pl.when(my == 3)(consumer(sched3))

    return pl.pallas_call(
        body,
        out_shape=jax.ShapeDtypeStruct((B, Sq, D), jnp.float32),
        in_specs=[pl.BlockSpec(memory_space=pltpu.VMEM)] * 5,
        out_specs=pl.BlockSpec(memory_space=pltpu.VMEM),
        scratch_shapes=[
            pltpu.VMEM((NS, Sq, Dh), jnp.bfloat16),
            pltpu.SemaphoreType.DMA((NS,)),
            pltpu.SemaphoreType.DMA((NS,)),
            pltpu.SemaphoreType.DMA((NS,)),
        ],
        compiler_params=pltpu.CompilerParams(collective_id=0),
    )(x, Wqh, Kt, Vt, Wo)
